# baseline (speedup 1.0000x reference)
import functools

import jax
import jax.numpy as jnp
from jax import lax
from jax.experimental import pallas as pl
from jax.experimental.pallas import tpu as pltpu
from jax.experimental.pallas import tpu_sc as plsc

DIM = 64
BATCH = 4096
NUM_ROWS = 1000000
TAIL_START = (NUM_ROWS // 128) * 128
TAIL = NUM_ROWS - TAIL_START
NUM_NEG = 8
NUM_CORES = 2
NUM_SUBCORES = 16
NUM_WORKERS = NUM_CORES * NUM_SUBCORES
BPW = BATCH // NUM_WORKERS
RPW = 2
BPR = BPW // RPW
GROUPS = BPR // 16


def _worker_id():
  return lax.axis_index("s") * NUM_CORES + lax.axis_index("c")


def _sc_users_body(users_h, uembT_h, ustage_h, idx_vm, ublk, tailbuf, u_loc,
                   sem):
  wid = _worker_id()
  base = wid * BPW

  pltpu.sync_copy(users_h.at[pl.ds(base, BPW)], idx_vm.at[pl.ds(0, BPW)])
  pltpu.sync_copy(uembT_h.at[:, :, pl.ds(TAIL_START, TAIL)], tailbuf)

  iota = lax.iota(jnp.int32, 16)
  half_a = iota >> 3
  mvec = iota & 7

  def scalar_idx(b):
    return idx_vm[pl.ds(b, 16)][0]

  def fire(b):
    j = scalar_idx(b)
    jc = jnp.minimum(j >> 7, TAIL_START // 128 - 1)
    off = pl.multiple_of(jc * 128, 128)
    p = b & 1
    for a in range(DIM // 8):
      pltpu.async_copy(uembT_h.at[a, :, pl.ds(off, 128)], ublk.at[p, a], sem)

  def drain():
    for _ in range(DIM // 8):
      pltpu.make_async_copy(uembT_h.at[0, :, pl.ds(0, 128)], ublk.at[0, 0],
                            sem).wait()

  fire(0)

  @pl.loop(0, BPW)
  def _per_index(b):
    @pl.when(b + 1 < BPW)
    def _():
      fire(b + 1)
    drain()
    j = scalar_idx(b)
    p = jnp.full((16,), b & 1, jnp.int32)
    col = jnp.full((16,), j & 127, jnp.int32)
    tcol = jnp.full((16,), jnp.maximum(j - TAIL_START, 0), jnp.int32)
    tmask = jnp.full((16,), j, jnp.int32) >= TAIL_START
    prow = jnp.full((16,), b >> 1, jnp.int32)
    pcol = iota + ((b & 1) << 6)
    for q in range(4):
      avec = half_a + 2 * q
      vn = plsc.load_gather(ublk, [p, avec, mvec, col])
      vt = plsc.load_gather(tailbuf, [avec, mvec, tcol])
      v = jnp.where(tmask, vt, vn)
      plsc.store_scatter(u_loc, [prow, pcol + 16 * q], v)

  pltpu.sync_copy(
      u_loc, ustage_h.at[pl.ds(pl.multiple_of(base // 2, 8), BPW // 2)])


_sc_users = functools.partial(
    pl.kernel,
    mesh=plsc.VectorSubcoreMesh(core_axis_name="c", subcore_axis_name="s"),
    out_type=jax.ShapeDtypeStruct((BATCH // 2, 2 * DIM), jnp.float32),
    scratch_types=[
        pltpu.VMEM((BPW + 16,), jnp.int32),
        pltpu.VMEM((2, DIM // 8, 8, 128), jnp.float32),
        pltpu.VMEM((DIM // 8, 8, TAIL), jnp.float32),
        pltpu.VMEM((BPW // 2, 2 * DIM), jnp.float32),
        pltpu.SemaphoreType.DMA,
    ],
    compiler_params=pltpu.CompilerParams(
        needs_layout_passes=False, use_tc_tiling_on_sc=True),
)(_sc_users_body)


def _sc_scores_body(items_h, negs_h, iemb2_h, ustage_h, pos_h, nexp_h,
                    i_idx, n_idx, i2, n2, u_loc, i_rows, n_rows, pos_v,
                    nexp_v, sem):
  wid = _worker_id()
  base = wid * BPW

  pltpu.sync_copy(items_h.at[pl.ds(base, BPW)], i_idx)
  for k in range(NUM_NEG):
    pltpu.sync_copy(negs_h.at[pl.ds(k * BATCH + base, BPW)], n_idx.at[k])

  iota = lax.iota(jnp.int32, 16)
  zero = jnp.zeros((16,), jnp.float32)
  one = jnp.full((16,), 1, jnp.int32)

  for r in range(RPW):
    for c in range(GROUPS):
      v = i_idx[pl.ds(r * BPR + 16 * c, 16)]
      i2[pl.ds(16 * c, 16)] = ((v >> 9) << 8) | (v & 255)
      for k in range(NUM_NEG):
        v = n_idx[k, pl.ds(r * BPR + 16 * c, 16)]
        n2[k, pl.ds(16 * c, 16)] = ((v >> 9) << 8) | (v & 255)

    pltpu.sync_copy(
        ustage_h.at[pl.ds(
            pl.multiple_of(base // 2 + r * (BPR // 2), 8), BPR // 2)], u_loc)
    cps = [pltpu.async_copy(iemb2_h.at[i2], i_rows, sem)]
    for k in range(NUM_NEG):
      cps.append(pltpu.async_copy(iemb2_h.at[n2.at[k]], n_rows.at[k], sem))
    for cp in cps:
      cp.wait()

    for g in range(GROUPS):
      row = iota + 16 * g
      lrow = row >> 1
      ucol = (row & one) << 6
      rf = iota + (r * BPR + 16 * g)
      icol = ((plsc.load_gather(i_idx, [rf]) >> 8) & one) << 6
      ncol = [
          ((plsc.load_gather(n_idx, [jnp.full((16,), k, jnp.int32), rf]) >> 8)
           & one) << 6 for k in range(NUM_NEG)
      ]

      def dim_body(d, carry, row=row, lrow=lrow, ucol=ucol, icol=icol,
                   ncol=ncol):
        ds = jnp.full((16,), d, jnp.int32)
        u_d = plsc.load_gather(u_loc, [lrow, ucol + ds])
        p = carry[0] + u_d * plsc.load_gather(i_rows, [row, icol + ds])
        ns = []
        for k in range(NUM_NEG):
          kv = jnp.full((16,), k, jnp.int32)
          ns.append(carry[1 + k] +
                    u_d * plsc.load_gather(n_rows, [kv, row, ncol[k] + ds]))
        return (p, *ns)

      scores = lax.fori_loop(0, DIM, dim_body, (zero,) * (1 + NUM_NEG))
      sl = pl.ds(r * BPR + 16 * g, 16)
      pos_v[sl] = scores[0]
      nexp = jnp.exp(scores[1])
      for k in range(2, NUM_NEG + 1):
        nexp = nexp + jnp.exp(scores[k])
      nexp_v[sl] = nexp

  pltpu.sync_copy(pos_v, pos_h.at[pl.ds(base, BPW)])
  pltpu.sync_copy(nexp_v, nexp_h.at[pl.ds(base, BPW)])


_sc_scores = functools.partial(
    pl.kernel,
    mesh=plsc.VectorSubcoreMesh(core_axis_name="c", subcore_axis_name="s"),
    out_type=[
        jax.ShapeDtypeStruct((BATCH,), jnp.float32),
        jax.ShapeDtypeStruct((BATCH,), jnp.float32),
    ],
    scratch_types=[
        pltpu.VMEM((BPW,), jnp.int32),
        pltpu.VMEM((NUM_NEG, BPW), jnp.int32),
        pltpu.VMEM((BPR,), jnp.int32),
        pltpu.VMEM((NUM_NEG, BPR), jnp.int32),
        pltpu.VMEM((BPR // 2, 2 * DIM), jnp.float32),
        pltpu.VMEM((BPR, 2 * DIM), jnp.float32),
        pltpu.VMEM((NUM_NEG, BPR, 2 * DIM), jnp.float32),
        pltpu.VMEM((BPW,), jnp.float32),
        pltpu.VMEM((BPW,), jnp.float32),
        pltpu.SemaphoreType.DMA,
    ],
    compiler_params=pltpu.CompilerParams(
        needs_layout_passes=False, use_tc_tiling_on_sc=True),
)(_sc_scores_body)


def _tc_transpose_body(xT_ref, o_ref):
  x = xT_ref[...]
  ii = lax.broadcasted_iota(jnp.int32, (256, 256), 0)
  jj = lax.broadcasted_iota(jnp.int32, (256, 256), 1)
  eye = jnp.where(ii == jj, 1.0, 0.0).astype(jnp.float32)
  dn = (((1,), (1,)), ((), ()))
  o_ref[:, 0:DIM] = lax.dot_general(
      eye, x[:, 0:256], dn, preferred_element_type=jnp.float32)
  o_ref[:, DIM:2 * DIM] = lax.dot_general(
      eye, x[:, 256:512], dn, preferred_element_type=jnp.float32)


_N_TBLK = 1954
_tc_transpose = pl.pallas_call(
    _tc_transpose_body,
    out_shape=jax.ShapeDtypeStruct((_N_TBLK * 256, 2 * DIM), jnp.float32),
    grid=(_N_TBLK,),
    in_specs=[pl.BlockSpec((DIM, 512), lambda c: (0, c))],
    out_specs=pl.BlockSpec((256, 2 * DIM), lambda c: (c, 0)),
)


def _tc_loss_body(pos_ref, nexp_ref, o_ref):
  pe = jnp.exp(pos_ref[...])
  ne = nexp_ref[...]
  losses = -jnp.log(pe / (pe + ne))
  o_ref[0, 0] = jnp.sum(losses) * (1.0 / BATCH)


_tc_loss = pl.pallas_call(
    _tc_loss_body,
    out_shape=jax.ShapeDtypeStruct((1, 1), jnp.float32),
    out_specs=pl.BlockSpec(memory_space=pltpu.SMEM),
)


def kernel(users, items, negatives, user_emb, item_emb):
  users = users.astype(jnp.int32)
  items = items.astype(jnp.int32)
  negatives = negatives.astype(jnp.int32)
  uembT3 = user_emb.T.reshape(DIM // 8, 8, NUM_ROWS)
  ustage = _sc_users(users, uembT3)
  iemb2 = _tc_transpose(item_emb.T)
  pos, nexp = _sc_scores(items, negatives, iemb2, ustage)
  out = _tc_loss(pos.reshape(32, 128), nexp.reshape(32, 128))
  return out[0, 0]


# speedup vs baseline: 1.0007x; 1.0007x over previous
import functools

import jax
import jax.numpy as jnp
from jax import lax
from jax.experimental import pallas as pl
from jax.experimental.pallas import tpu as pltpu
from jax.experimental.pallas import tpu_sc as plsc

DIM = 64
BATCH = 4096
NUM_ROWS = 1000000
TAIL_START = (NUM_ROWS // 128) * 128
TAIL = NUM_ROWS - TAIL_START
NUM_NEG = 8
NUM_CORES = 2
NUM_SUBCORES = 16
NUM_WORKERS = NUM_CORES * NUM_SUBCORES
BPW = BATCH // NUM_WORKERS
RPW = 2
BPR = BPW // RPW
GROUPS = BPR // 16


def _worker_id():
  return lax.axis_index("s") * NUM_CORES + lax.axis_index("c")


def _sc_users_body(users_h, uembT_h, ustage_h, idx_vm, ublk, tailbuf, u_loc,
                   sem):
  wid = _worker_id()
  base = wid * BPW

  pltpu.sync_copy(users_h.at[pl.ds(base, BPW)], idx_vm.at[pl.ds(0, BPW)])
  pltpu.sync_copy(uembT_h.at[:, :, pl.ds(TAIL_START, TAIL)], tailbuf)

  iota = lax.iota(jnp.int32, 16)
  half_a = iota >> 3
  mvec = iota & 7

  def scalar_idx(b):
    return idx_vm[pl.ds(b, 16)][0]

  def fire(b):
    j = scalar_idx(b)
    jc = jnp.minimum(j >> 7, TAIL_START // 128 - 1)
    off = pl.multiple_of(jc * 128, 128)
    p = b & 1
    for a in range(DIM // 8):
      pltpu.async_copy(uembT_h.at[a, :, pl.ds(off, 128)], ublk.at[p, a], sem)

  def drain():
    for _ in range(DIM // 8):
      pltpu.make_async_copy(uembT_h.at[0, :, pl.ds(0, 128)], ublk.at[0, 0],
                            sem).wait()

  fire(0)

  @pl.loop(0, BPW)
  def _per_index(b):
    @pl.when(b + 1 < BPW)
    def _():
      fire(b + 1)
    drain()
    j = scalar_idx(b)
    p = jnp.full((16,), b & 1, jnp.int32)
    col = jnp.full((16,), j & 127, jnp.int32)
    tcol = jnp.full((16,), jnp.maximum(j - TAIL_START, 0), jnp.int32)
    tmask = jnp.full((16,), j, jnp.int32) >= TAIL_START
    prow = jnp.full((16,), b >> 1, jnp.int32)
    pcol = iota + ((b & 1) << 6)
    for q in range(4):
      avec = half_a + 2 * q
      vn = plsc.load_gather(ublk, [p, avec, mvec, col])
      vt = plsc.load_gather(tailbuf, [avec, mvec, tcol])
      v = jnp.where(tmask, vt, vn)
      plsc.store_scatter(u_loc, [prow, pcol + 16 * q], v)

  pltpu.sync_copy(
      u_loc, ustage_h.at[pl.ds(pl.multiple_of(base // 2, 8), BPW // 2)])


_sc_users = functools.partial(
    pl.kernel,
    mesh=plsc.VectorSubcoreMesh(core_axis_name="c", subcore_axis_name="s"),
    out_type=jax.ShapeDtypeStruct((BATCH // 2, 2 * DIM), jnp.float32),
    scratch_types=[
        pltpu.VMEM((BPW + 16,), jnp.int32),
        pltpu.VMEM((2, DIM // 8, 8, 128), jnp.float32),
        pltpu.VMEM((DIM // 8, 8, TAIL), jnp.float32),
        pltpu.VMEM((BPW // 2, 2 * DIM), jnp.float32),
        pltpu.SemaphoreType.DMA,
    ],
    compiler_params=pltpu.CompilerParams(
        needs_layout_passes=False, use_tc_tiling_on_sc=True),
)(_sc_users_body)


def _sc_scores_body(items_h, negs_h, iemb2_h, ustage_h, pos_h, nexp_h,
                    i_idx, n_idx, i2, n2, u_loc, i_rows, n_rows, pos_v,
                    nexp_v, sem):
  wid = _worker_id()
  base = wid * BPW

  pltpu.sync_copy(items_h.at[pl.ds(base, BPW)], i_idx)
  for k in range(NUM_NEG):
    pltpu.sync_copy(negs_h.at[pl.ds(k * BATCH + base, BPW)], n_idx.at[k])

  iota = lax.iota(jnp.int32, 16)
  zero = jnp.zeros((16,), jnp.float32)
  one = jnp.full((16,), 1, jnp.int32)

  for r in range(RPW):
    for c in range(GROUPS):
      v = i_idx[pl.ds(r * BPR + 16 * c, 16)]
      i2[pl.ds(16 * c, 16)] = ((v >> 9) << 8) | (v & 255)
      for k in range(NUM_NEG):
        v = n_idx[k, pl.ds(r * BPR + 16 * c, 16)]
        n2[k, pl.ds(16 * c, 16)] = ((v >> 9) << 8) | (v & 255)

    pltpu.sync_copy(
        ustage_h.at[pl.ds(
            pl.multiple_of(base // 2 + r * (BPR // 2), 8), BPR // 2)], u_loc)
    cps = [pltpu.async_copy(iemb2_h.at[i2], i_rows, sem)]
    for k in range(NUM_NEG):
      cps.append(pltpu.async_copy(iemb2_h.at[n2.at[k]], n_rows.at[k], sem))
    for cp in cps:
      cp.wait()

    for g in range(GROUPS):
      row = iota + 16 * g
      lrow = row >> 1
      ucol = (row & one) << 6
      rf = iota + (r * BPR + 16 * g)
      icol = ((plsc.load_gather(i_idx, [rf]) >> 8) & one) << 6
      ncol = [
          ((plsc.load_gather(n_idx, [jnp.full((16,), k, jnp.int32), rf]) >> 8)
           & one) << 6 for k in range(NUM_NEG)
      ]

      def dim_body(d, carry, row=row, lrow=lrow, ucol=ucol, icol=icol,
                   ncol=ncol):
        ds = jnp.full((16,), d, jnp.int32)
        u_d = plsc.load_gather(u_loc, [lrow, ucol + ds])
        p = carry[0] + u_d * plsc.load_gather(i_rows, [row, icol + ds])
        ns = []
        for k in range(NUM_NEG):
          kv = jnp.full((16,), k, jnp.int32)
          ns.append(carry[1 + k] +
                    u_d * plsc.load_gather(n_rows, [kv, row, ncol[k] + ds]))
        return (p, *ns)

      scores = lax.fori_loop(0, DIM, dim_body, (zero,) * (1 + NUM_NEG))
      sl = pl.ds(r * BPR + 16 * g, 16)
      pos_v[sl] = scores[0]
      nexp = jnp.exp(scores[1])
      for k in range(2, NUM_NEG + 1):
        nexp = nexp + jnp.exp(scores[k])
      nexp_v[sl] = nexp

  pltpu.sync_copy(pos_v, pos_h.at[pl.ds(base, BPW)])
  pltpu.sync_copy(nexp_v, nexp_h.at[pl.ds(base, BPW)])


_sc_scores = functools.partial(
    pl.kernel,
    mesh=plsc.VectorSubcoreMesh(core_axis_name="c", subcore_axis_name="s"),
    out_type=[
        jax.ShapeDtypeStruct((BATCH,), jnp.float32),
        jax.ShapeDtypeStruct((BATCH,), jnp.float32),
    ],
    scratch_types=[
        pltpu.VMEM((BPW,), jnp.int32),
        pltpu.VMEM((NUM_NEG, BPW), jnp.int32),
        pltpu.VMEM((BPR,), jnp.int32),
        pltpu.VMEM((NUM_NEG, BPR), jnp.int32),
        pltpu.VMEM((BPR // 2, 2 * DIM), jnp.float32),
        pltpu.VMEM((BPR, 2 * DIM), jnp.float32),
        pltpu.VMEM((NUM_NEG, BPR, 2 * DIM), jnp.float32),
        pltpu.VMEM((BPW,), jnp.float32),
        pltpu.VMEM((BPW,), jnp.float32),
        pltpu.SemaphoreType.DMA,
    ],
    compiler_params=pltpu.CompilerParams(
        needs_layout_passes=False, use_tc_tiling_on_sc=True),
)(_sc_scores_body)


def _tc_transpose_body(xT_ref, eye_ref, o_ref):
  x = xT_ref[...]
  eye = eye_ref[...]
  dn = (((1,), (1,)), ((), ()))
  o_ref[:, 0:DIM] = lax.dot_general(
      eye, x[:, 0:256], dn, preferred_element_type=jnp.float32)
  o_ref[:, DIM:2 * DIM] = lax.dot_general(
      eye, x[:, 256:512], dn, preferred_element_type=jnp.float32)


_N_TBLK = 1954
_tc_transpose = pl.pallas_call(
    _tc_transpose_body,
    out_shape=jax.ShapeDtypeStruct((_N_TBLK * 256, 2 * DIM), jnp.float32),
    grid=(_N_TBLK,),
    in_specs=[
        pl.BlockSpec((DIM, 512), lambda c: (0, c)),
        pl.BlockSpec((256, 256), lambda c: (0, 0)),
    ],
    out_specs=pl.BlockSpec((256, 2 * DIM), lambda c: (c, 0)),
)


def _tc_loss_body(pos_ref, nexp_ref, o_ref):
  pe = jnp.exp(pos_ref[...])
  ne = nexp_ref[...]
  losses = -jnp.log(pe / (pe + ne))
  o_ref[0, 0] = jnp.sum(losses) * (1.0 / BATCH)


_tc_loss = pl.pallas_call(
    _tc_loss_body,
    out_shape=jax.ShapeDtypeStruct((1, 1), jnp.float32),
    out_specs=pl.BlockSpec(memory_space=pltpu.SMEM),
)


def kernel(users, items, negatives, user_emb, item_emb):
  users = users.astype(jnp.int32)
  items = items.astype(jnp.int32)
  negatives = negatives.astype(jnp.int32)
  uembT3 = user_emb.T.reshape(DIM // 8, 8, NUM_ROWS)
  ustage = _sc_users(users, uembT3)
  iemb2 = _tc_transpose(item_emb.T, jnp.eye(256, dtype=jnp.float32))
  pos, nexp = _sc_scores(items, negatives, iemb2, ustage)
  out = _tc_loss(pos.reshape(32, 128), nexp.reshape(32, 128))
  return out[0, 0]


# speedup vs baseline: 2.7233x; 2.7215x over previous
import functools

import jax
import jax.numpy as jnp
from jax import lax
from jax.experimental import pallas as pl
from jax.experimental.pallas import tpu as pltpu
from jax.experimental.pallas import tpu_sc as plsc

DIM = 64
BATCH = 4096
NUM_ROWS = 1000000
TAIL_START = (NUM_ROWS // 128) * 128
TAIL = NUM_ROWS - TAIL_START
NUM_NEG = 8
NUM_CORES = 2
NUM_SUBCORES = 16
NUM_WORKERS = NUM_CORES * NUM_SUBCORES
BPW = BATCH // NUM_WORKERS
RPW = 2
BPR = BPW // RPW
GROUPS = BPR // 16


def _worker_id():
  return lax.axis_index("s") * NUM_CORES + lax.axis_index("c")


def _sc_users_body(users_h, uembT_h, ustage_h, idx_vm, ublk, tailbuf, u_loc,
                   sem):
  wid = _worker_id()
  base = wid * BPW

  pltpu.sync_copy(users_h.at[pl.ds(base, BPW)], idx_vm.at[pl.ds(0, BPW)])
  pltpu.sync_copy(uembT_h.at[:, :, pl.ds(TAIL_START, TAIL)], tailbuf)

  iota = lax.iota(jnp.int32, 16)
  half_a = iota >> 3
  mvec = iota & 7

  def scalar_idx(b):
    return idx_vm[pl.ds(b, 16)][0]

  def fire(b):
    j = scalar_idx(b)
    jc = jnp.minimum(j >> 7, TAIL_START // 128 - 1)
    off = pl.multiple_of(jc * 128, 128)
    p = b & 1
    for a in range(DIM // 8):
      pltpu.async_copy(uembT_h.at[a, :, pl.ds(off, 128)], ublk.at[p, a], sem)

  def drain():
    for _ in range(DIM // 8):
      pltpu.make_async_copy(uembT_h.at[0, :, pl.ds(0, 128)], ublk.at[0, 0],
                            sem).wait()

  fire(0)

  @pl.loop(0, BPW)
  def _per_index(b):
    @pl.when(b + 1 < BPW)
    def _():
      fire(b + 1)
    drain()
    j = scalar_idx(b)
    p = jnp.full((16,), b & 1, jnp.int32)
    col = jnp.full((16,), j & 127, jnp.int32)
    tcol = jnp.full((16,), jnp.maximum(j - TAIL_START, 0), jnp.int32)
    tmask = jnp.full((16,), j, jnp.int32) >= TAIL_START
    prow = jnp.full((16,), b >> 1, jnp.int32)
    pcol = iota + ((b & 1) << 6)
    for q in range(4):
      avec = half_a + 2 * q
      vn = plsc.load_gather(ublk, [p, avec, mvec, col])
      vt = plsc.load_gather(tailbuf, [avec, mvec, tcol])
      v = jnp.where(tmask, vt, vn)
      plsc.store_scatter(u_loc, [prow, pcol + 16 * q], v)

  pltpu.sync_copy(
      u_loc, ustage_h.at[pl.ds(pl.multiple_of(base // 2, 8), BPW // 2)])


_sc_users = functools.partial(
    pl.kernel,
    mesh=plsc.VectorSubcoreMesh(core_axis_name="c", subcore_axis_name="s"),
    out_type=jax.ShapeDtypeStruct((BATCH // 2, 2 * DIM), jnp.float32),
    scratch_types=[
        pltpu.VMEM((BPW + 16,), jnp.int32),
        pltpu.VMEM((2, DIM // 8, 8, 128), jnp.float32),
        pltpu.VMEM((DIM // 8, 8, TAIL), jnp.float32),
        pltpu.VMEM((BPW // 2, 2 * DIM), jnp.float32),
        pltpu.SemaphoreType.DMA,
    ],
    compiler_params=pltpu.CompilerParams(
        needs_layout_passes=False, use_tc_tiling_on_sc=True),
)(_sc_users_body)


def _sc_scores_body(items_h, negs_h, iemb2_h, ustage_h, pos_h, nexp_h,
                    i_idx, n_idx, i2, n2, u_loc, i_rows, n_rows, pos_v,
                    nexp_v, sem):
  wid = _worker_id()
  base = wid * BPW

  pltpu.sync_copy(items_h.at[pl.ds(base, BPW)], i_idx)
  for k in range(NUM_NEG):
    pltpu.sync_copy(negs_h.at[pl.ds(k * BATCH + base, BPW)], n_idx.at[k])

  iota = lax.iota(jnp.int32, 16)
  zero = jnp.zeros((16,), jnp.float32)
  one = jnp.full((16,), 1, jnp.int32)

  for r in range(RPW):
    for c in range(GROUPS):
      v = i_idx[pl.ds(r * BPR + 16 * c, 16)]
      i2[pl.ds(16 * c, 16)] = ((v >> 9) << 8) | (v & 255)
      for k in range(NUM_NEG):
        v = n_idx[k, pl.ds(r * BPR + 16 * c, 16)]
        n2[k, pl.ds(16 * c, 16)] = ((v >> 9) << 8) | (v & 255)

    pltpu.sync_copy(
        ustage_h.at[pl.ds(
            pl.multiple_of(base // 2 + r * (BPR // 2), 8), BPR // 2)], u_loc)
    cps = [pltpu.async_copy(iemb2_h.at[i2], i_rows, sem)]
    for k in range(NUM_NEG):
      cps.append(pltpu.async_copy(iemb2_h.at[n2.at[k]], n_rows.at[k], sem))
    for cp in cps:
      cp.wait()

    for g in range(GROUPS):
      row = iota + 16 * g
      lrow = row >> 1
      ucol = (row & one) << 6
      rf = iota + (r * BPR + 16 * g)
      icol = ((plsc.load_gather(i_idx, [rf]) >> 8) & one) << 6
      ncol = [
          ((plsc.load_gather(n_idx, [jnp.full((16,), k, jnp.int32), rf]) >> 8)
           & one) << 6 for k in range(NUM_NEG)
      ]

      def dim_body(d, carry, row=row, lrow=lrow, ucol=ucol, icol=icol,
                   ncol=ncol):
        ds = jnp.full((16,), d, jnp.int32)
        u_d = plsc.load_gather(u_loc, [lrow, ucol + ds])
        p = carry[0] + u_d * plsc.load_gather(i_rows, [row, icol + ds])
        ns = []
        for k in range(NUM_NEG):
          kv = jnp.full((16,), k, jnp.int32)
          ns.append(carry[1 + k] +
                    u_d * plsc.load_gather(n_rows, [kv, row, ncol[k] + ds]))
        return (p, *ns)

      scores = lax.fori_loop(0, DIM, dim_body, (zero,) * (1 + NUM_NEG))
      sl = pl.ds(r * BPR + 16 * g, 16)
      pos_v[sl] = scores[0]
      nexp = jnp.exp(scores[1])
      for k in range(2, NUM_NEG + 1):
        nexp = nexp + jnp.exp(scores[k])
      nexp_v[sl] = nexp

  pltpu.sync_copy(pos_v, pos_h.at[pl.ds(base, BPW)])
  pltpu.sync_copy(nexp_v, nexp_h.at[pl.ds(base, BPW)])


_sc_scores = functools.partial(
    pl.kernel,
    mesh=plsc.VectorSubcoreMesh(core_axis_name="c", subcore_axis_name="s"),
    out_type=[
        jax.ShapeDtypeStruct((BATCH,), jnp.float32),
        jax.ShapeDtypeStruct((BATCH,), jnp.float32),
    ],
    scratch_types=[
        pltpu.VMEM((BPW,), jnp.int32),
        pltpu.VMEM((NUM_NEG, BPW), jnp.int32),
        pltpu.VMEM((BPR,), jnp.int32),
        pltpu.VMEM((NUM_NEG, BPR), jnp.int32),
        pltpu.VMEM((BPR // 2, 2 * DIM), jnp.float32),
        pltpu.VMEM((BPR, 2 * DIM), jnp.float32),
        pltpu.VMEM((NUM_NEG, BPR, 2 * DIM), jnp.float32),
        pltpu.VMEM((BPW,), jnp.float32),
        pltpu.VMEM((BPW,), jnp.float32),
        pltpu.SemaphoreType.DMA,
    ],
    compiler_params=pltpu.CompilerParams(
        needs_layout_passes=False, use_tc_tiling_on_sc=True),
)(_sc_scores_body)


def _tc_transpose_body(xT_ref, eye_ref, o_ref):
  eye = eye_ref[...]
  dn = (((1,), (1,)), ((), ()))
  for m in range(4):
    x = xT_ref[:, pl.ds(512 * m, 512)]
    xx = jnp.concatenate([x[:, 0:256], x[:, 256:512]], axis=0)
    o_ref[pl.ds(256 * m, 256), :] = lax.dot_general(
        eye, xx, dn, preferred_element_type=jnp.float32)


_N_TBLK = 489
_tc_transpose = pl.pallas_call(
    _tc_transpose_body,
    out_shape=jax.ShapeDtypeStruct((_N_TBLK * 1024, 2 * DIM), jnp.float32),
    grid=(_N_TBLK,),
    in_specs=[
        pl.BlockSpec((DIM, 2048), lambda c: (0, c)),
        pl.BlockSpec((256, 256), lambda c: (0, 0)),
    ],
    out_specs=pl.BlockSpec((1024, 2 * DIM), lambda c: (c, 0)),
)


def _tc_loss_body(pos_ref, nexp_ref, o_ref):
  pe = jnp.exp(pos_ref[...])
  ne = nexp_ref[...]
  losses = -jnp.log(pe / (pe + ne))
  o_ref[0, 0] = jnp.sum(losses) * (1.0 / BATCH)


_tc_loss = pl.pallas_call(
    _tc_loss_body,
    out_shape=jax.ShapeDtypeStruct((1, 1), jnp.float32),
    out_specs=pl.BlockSpec(memory_space=pltpu.SMEM),
)


def kernel(users, items, negatives, user_emb, item_emb):
  users = users.astype(jnp.int32)
  items = items.astype(jnp.int32)
  negatives = negatives.astype(jnp.int32)
  uembT3 = user_emb.T.reshape(DIM // 8, 8, NUM_ROWS)
  ustage = _sc_users(users, uembT3)
  iemb2 = _tc_transpose(item_emb.T, jnp.eye(256, dtype=jnp.float32))
  pos, nexp = _sc_scores(items, negatives, iemb2, ustage)
  out = _tc_loss(pos.reshape(32, 128), nexp.reshape(32, 128))
  return out[0, 0]


# speedup vs baseline: 2.7490x; 1.0095x over previous
import functools

import jax
import jax.numpy as jnp
from jax import lax
from jax.experimental import pallas as pl
from jax.experimental.pallas import tpu as pltpu
from jax.experimental.pallas import tpu_sc as plsc

DIM = 64
BATCH = 4096
NUM_ROWS = 1000000
TAIL_START = (NUM_ROWS // 128) * 128
TAIL = NUM_ROWS - TAIL_START
NUM_NEG = 8
NUM_CORES = 2
NUM_SUBCORES = 16
NUM_WORKERS = NUM_CORES * NUM_SUBCORES
BPW = BATCH // NUM_WORKERS
RPW = 2
BPR = BPW // RPW
GROUPS = BPR // 16


def _worker_id():
  return lax.axis_index("s") * NUM_CORES + lax.axis_index("c")


def _sc_users_body(users_h, uembT_h, ustage_h, idx_vm, ublk, tailbuf, u_loc,
                   sem):
  wid = _worker_id()
  base = wid * BPW

  pltpu.sync_copy(users_h.at[pl.ds(base, BPW)], idx_vm.at[pl.ds(0, BPW)])
  pltpu.sync_copy(uembT_h.at[:, :, pl.ds(TAIL_START, TAIL)], tailbuf)

  iota = lax.iota(jnp.int32, 16)
  half_a = iota >> 3
  mvec = iota & 7

  def scalar_idx(b):
    return idx_vm[pl.ds(b, 16)][0]

  def fire(b):
    j = scalar_idx(b)
    jc = jnp.minimum(j >> 7, TAIL_START // 128 - 1)
    off = pl.multiple_of(jc * 128, 128)
    p = b & 1
    for a in range(DIM // 8):
      pltpu.async_copy(uembT_h.at[a, :, pl.ds(off, 128)], ublk.at[p, a], sem)

  def drain():
    for _ in range(DIM // 8):
      pltpu.make_async_copy(uembT_h.at[0, :, pl.ds(0, 128)], ublk.at[0, 0],
                            sem).wait()

  fire(0)

  @pl.loop(0, BPW)
  def _per_index(b):
    @pl.when(b + 1 < BPW)
    def _():
      fire(b + 1)
    drain()
    j = scalar_idx(b)
    p = jnp.full((16,), b & 1, jnp.int32)
    col = jnp.full((16,), j & 127, jnp.int32)
    tcol = jnp.full((16,), jnp.maximum(j - TAIL_START, 0), jnp.int32)
    tmask = jnp.full((16,), j, jnp.int32) >= TAIL_START
    prow = jnp.full((16,), b >> 1, jnp.int32)
    pcol = iota + ((b & 1) << 6)
    for q in range(4):
      avec = half_a + 2 * q
      vn = plsc.load_gather(ublk, [p, avec, mvec, col])
      vt = plsc.load_gather(tailbuf, [avec, mvec, tcol])
      v = jnp.where(tmask, vt, vn)
      plsc.store_scatter(u_loc, [prow, pcol + 16 * q], v)

  pltpu.sync_copy(
      u_loc, ustage_h.at[pl.ds(pl.multiple_of(base // 2, 8), BPW // 2)])


_sc_users = functools.partial(
    pl.kernel,
    mesh=plsc.VectorSubcoreMesh(core_axis_name="c", subcore_axis_name="s"),
    out_type=jax.ShapeDtypeStruct((BATCH // 2, 2 * DIM), jnp.float32),
    scratch_types=[
        pltpu.VMEM((BPW + 16,), jnp.int32),
        pltpu.VMEM((2, DIM // 8, 8, 128), jnp.float32),
        pltpu.VMEM((DIM // 8, 8, TAIL), jnp.float32),
        pltpu.VMEM((BPW // 2, 2 * DIM), jnp.float32),
        pltpu.SemaphoreType.DMA,
    ],
    compiler_params=pltpu.CompilerParams(
        needs_layout_passes=False, use_tc_tiling_on_sc=True),
)(_sc_users_body)


def _sc_scores_body(items_h, negs_h, iemb2_h, ustage_h, pos_h, nexp_h,
                    i_idx, n_idx, i2, n2, u_loc, i_rows, n_rows, pos_v,
                    nexp_v, sem):
  wid = _worker_id()
  base = wid * BPW

  pltpu.sync_copy(items_h.at[pl.ds(base, BPW)], i_idx)
  for k in range(NUM_NEG):
    pltpu.sync_copy(negs_h.at[pl.ds(k * BATCH + base, BPW)], n_idx.at[k])

  iota = lax.iota(jnp.int32, 16)
  zero = jnp.zeros((16,), jnp.float32)
  one = jnp.full((16,), 1, jnp.int32)

  for r in range(RPW):
    for c in range(GROUPS):
      v = i_idx[pl.ds(r * BPR + 16 * c, 16)]
      i2[pl.ds(16 * c, 16)] = ((v >> 9) << 8) | (v & 255)
      for k in range(NUM_NEG):
        v = n_idx[k, pl.ds(r * BPR + 16 * c, 16)]
        n2[k, pl.ds(16 * c, 16)] = ((v >> 9) << 8) | (v & 255)

    pltpu.sync_copy(
        ustage_h.at[pl.ds(
            pl.multiple_of(base // 2 + r * (BPR // 2), 8), BPR // 2)], u_loc)
    cps = [pltpu.async_copy(iemb2_h.at[i2], i_rows, sem)]
    for k in range(NUM_NEG):
      cps.append(pltpu.async_copy(iemb2_h.at[n2.at[k]], n_rows.at[k], sem))
    for cp in cps:
      cp.wait()

    for g in range(GROUPS):
      row = iota + 16 * g
      lrow = row >> 1
      ucol = (row & one) << 6
      rf = iota + (r * BPR + 16 * g)
      icol = ((plsc.load_gather(i_idx, [rf]) >> 8) & one) << 6
      ncol = [
          ((plsc.load_gather(n_idx, [jnp.full((16,), k, jnp.int32), rf]) >> 8)
           & one) << 6 for k in range(NUM_NEG)
      ]

      kvs = [jnp.full((16,), k, jnp.int32) for k in range(NUM_NEG)]

      def dim_body(d, carry, row=row, lrow=lrow, ucol=ucol, icol=icol,
                   ncol=ncol, kvs=kvs):
        ds = jnp.full((16,), d, jnp.int32)
        u_d = plsc.load_gather(u_loc, [lrow, ucol + ds])
        p = carry[0] + u_d * plsc.load_gather(i_rows, [row, icol + ds])
        ns = []
        for k in range(NUM_NEG):
          ns.append(carry[1 + k] + u_d *
                    plsc.load_gather(n_rows, [kvs[k], row, ncol[k] + ds]))
        return (p, *ns)

      scores = lax.fori_loop(0, DIM, dim_body, (zero,) * (1 + NUM_NEG),
                             unroll=4)
      sl = pl.ds(r * BPR + 16 * g, 16)
      pos_v[sl] = scores[0]
      nexp = jnp.exp(scores[1])
      for k in range(2, NUM_NEG + 1):
        nexp = nexp + jnp.exp(scores[k])
      nexp_v[sl] = nexp

  pltpu.sync_copy(pos_v, pos_h.at[pl.ds(base, BPW)])
  pltpu.sync_copy(nexp_v, nexp_h.at[pl.ds(base, BPW)])


_sc_scores = functools.partial(
    pl.kernel,
    mesh=plsc.VectorSubcoreMesh(core_axis_name="c", subcore_axis_name="s"),
    out_type=[
        jax.ShapeDtypeStruct((BATCH,), jnp.float32),
        jax.ShapeDtypeStruct((BATCH,), jnp.float32),
    ],
    scratch_types=[
        pltpu.VMEM((BPW,), jnp.int32),
        pltpu.VMEM((NUM_NEG, BPW), jnp.int32),
        pltpu.VMEM((BPR,), jnp.int32),
        pltpu.VMEM((NUM_NEG, BPR), jnp.int32),
        pltpu.VMEM((BPR // 2, 2 * DIM), jnp.float32),
        pltpu.VMEM((BPR, 2 * DIM), jnp.float32),
        pltpu.VMEM((NUM_NEG, BPR, 2 * DIM), jnp.float32),
        pltpu.VMEM((BPW,), jnp.float32),
        pltpu.VMEM((BPW,), jnp.float32),
        pltpu.SemaphoreType.DMA,
    ],
    compiler_params=pltpu.CompilerParams(
        needs_layout_passes=False, use_tc_tiling_on_sc=True),
)(_sc_scores_body)


def _tc_transpose_body(xT_ref, eye_ref, o_ref):
  eye = eye_ref[...]
  dn = (((1,), (1,)), ((), ()))
  for m in range(4):
    x = xT_ref[:, pl.ds(512 * m, 512)]
    xx = jnp.concatenate([x[:, 0:256], x[:, 256:512]], axis=0)
    o_ref[pl.ds(256 * m, 256), :] = lax.dot_general(
        eye, xx, dn, preferred_element_type=jnp.float32)


_N_TBLK = 489
_tc_transpose = pl.pallas_call(
    _tc_transpose_body,
    out_shape=jax.ShapeDtypeStruct((_N_TBLK * 1024, 2 * DIM), jnp.float32),
    grid=(_N_TBLK,),
    in_specs=[
        pl.BlockSpec((DIM, 2048), lambda c: (0, c)),
        pl.BlockSpec((256, 256), lambda c: (0, 0)),
    ],
    out_specs=pl.BlockSpec((1024, 2 * DIM), lambda c: (c, 0)),
)


def _tc_loss_body(pos_ref, nexp_ref, o_ref):
  pe = jnp.exp(pos_ref[...])
  ne = nexp_ref[...]
  losses = -jnp.log(pe / (pe + ne))
  o_ref[0, 0] = jnp.sum(losses) * (1.0 / BATCH)


_tc_loss = pl.pallas_call(
    _tc_loss_body,
    out_shape=jax.ShapeDtypeStruct((1, 1), jnp.float32),
    out_specs=pl.BlockSpec(memory_space=pltpu.SMEM),
)


def kernel(users, items, negatives, user_emb, item_emb):
  users = users.astype(jnp.int32)
  items = items.astype(jnp.int32)
  negatives = negatives.astype(jnp.int32)
  uembT3 = user_emb.T.reshape(DIM // 8, 8, NUM_ROWS)
  ustage = _sc_users(users, uembT3)
  iemb2 = _tc_transpose(item_emb.T, jnp.eye(256, dtype=jnp.float32))
  pos, nexp = _sc_scores(items, negatives, iemb2, ustage)
  out = _tc_loss(pos.reshape(32, 128), nexp.reshape(32, 128))
  return out[0, 0]


# speedup vs baseline: 3.7181x; 1.3525x over previous
import functools

import jax
import jax.numpy as jnp
from jax import lax
from jax.experimental import pallas as pl
from jax.experimental.pallas import tpu as pltpu
from jax.experimental.pallas import tpu_sc as plsc

DIM = 64
BATCH = 4096
NUM_ROWS = 1000000
TAIL_START = (NUM_ROWS // 128) * 128
TAIL = NUM_ROWS - TAIL_START
NUM_NEG = 8
NUM_CORES = 2
NUM_SUBCORES = 16
NUM_WORKERS = NUM_CORES * NUM_SUBCORES
BPW = BATCH // NUM_WORKERS
RPW = 2
BPR = BPW // RPW
GROUPS = BPR // 16


def _worker_id():
  return lax.axis_index("s") * NUM_CORES + lax.axis_index("c")


def _sc_users_body(users_h, uembT_h, ustage_h, idx_vm, ublk, tailbuf, u_loc,
                   sem):
  wid = _worker_id()
  base = wid * BPW

  pltpu.sync_copy(users_h.at[pl.ds(base, BPW)], idx_vm.at[pl.ds(0, BPW)])
  pltpu.sync_copy(uembT_h.at[:, :, pl.ds(TAIL_START, TAIL)], tailbuf)

  iota = lax.iota(jnp.int32, 16)
  half_a = iota >> 3
  mvec = iota & 7

  def scalar_idx(b):
    return idx_vm[pl.ds(b, 16)][0]

  def fire(b):
    j = scalar_idx(b)
    jc = jnp.minimum(j >> 7, TAIL_START // 128 - 1)
    off = pl.multiple_of(jc * 128, 128)
    p = b & 1
    for a in range(DIM // 8):
      pltpu.async_copy(uembT_h.at[a, :, pl.ds(off, 128)], ublk.at[p, a], sem)

  def drain():
    for _ in range(DIM // 8):
      pltpu.make_async_copy(uembT_h.at[0, :, pl.ds(0, 128)], ublk.at[0, 0],
                            sem).wait()

  fire(0)

  @pl.loop(0, BPW)
  def _per_index(b):
    @pl.when(b + 1 < BPW)
    def _():
      fire(b + 1)
    drain()
    j = scalar_idx(b)
    p = jnp.full((16,), b & 1, jnp.int32)
    col = jnp.full((16,), j & 127, jnp.int32)
    tcol = jnp.full((16,), jnp.maximum(j - TAIL_START, 0), jnp.int32)
    tmask = jnp.full((16,), j, jnp.int32) >= TAIL_START
    prow = jnp.full((16,), b >> 1, jnp.int32)
    pcol = iota + ((b & 1) << 6)
    for q in range(4):
      avec = half_a + 2 * q
      vn = plsc.load_gather(ublk, [p, avec, mvec, col])
      vt = plsc.load_gather(tailbuf, [avec, mvec, tcol])
      v = jnp.where(tmask, vt, vn)
      plsc.store_scatter(u_loc, [prow, pcol + 16 * q], v)

  pltpu.sync_copy(
      u_loc, ustage_h.at[pl.ds(pl.multiple_of(base // 2, 8), BPW // 2)])


_sc_users = functools.partial(
    pl.kernel,
    mesh=plsc.VectorSubcoreMesh(core_axis_name="c", subcore_axis_name="s"),
    out_type=jax.ShapeDtypeStruct((BATCH // 2, 2 * DIM), jnp.float32),
    scratch_types=[
        pltpu.VMEM((BPW + 16,), jnp.int32),
        pltpu.VMEM((2, DIM // 8, 8, 128), jnp.float32),
        pltpu.VMEM((DIM // 8, 8, TAIL), jnp.float32),
        pltpu.VMEM((BPW // 2, 2 * DIM), jnp.float32),
        pltpu.SemaphoreType.DMA,
    ],
    compiler_params=pltpu.CompilerParams(
        needs_layout_passes=False, use_tc_tiling_on_sc=True),
)(_sc_users_body)


def _sc_scores_body(items_h, negs_h, iemb2_h, ustage_h, pos_h, nexp_h,
                    i_idx, n_idx, i2, n2, u_loc, i_rows, n_rows, pos_v,
                    nexp_v, sem):
  wid = _worker_id()
  base = wid * BPW

  pltpu.sync_copy(items_h.at[pl.ds(base, BPW)], i_idx)
  for k in range(NUM_NEG):
    pltpu.sync_copy(negs_h.at[pl.ds(k * BATCH + base, BPW)], n_idx.at[k])

  iota = lax.iota(jnp.int32, 16)
  zero = jnp.zeros((16,), jnp.float32)
  one = jnp.full((16,), 1, jnp.int32)

  for r in range(RPW):
    for c in range(GROUPS):
      v = i_idx[pl.ds(r * BPR + 16 * c, 16)]
      i2[pl.ds(16 * c, 16)] = ((v >> 9) << 8) | (v & 255)
      for k in range(NUM_NEG):
        v = n_idx[k, pl.ds(r * BPR + 16 * c, 16)]
        n2[k, pl.ds(16 * c, 16)] = ((v >> 9) << 8) | (v & 255)

    pltpu.sync_copy(
        ustage_h.at[pl.ds(
            pl.multiple_of(base // 2 + r * (BPR // 2), 8), BPR // 2)], u_loc)
    cps = [pltpu.async_copy(iemb2_h.at[i2], i_rows, sem)]
    for k in range(NUM_NEG):
      cps.append(pltpu.async_copy(iemb2_h.at[n2.at[k]], n_rows.at[k], sem))
    for cp in cps:
      cp.wait()

    for g in range(GROUPS):
      row = iota + 16 * g
      lrow = row >> 1
      ucol = (row & one) << 6
      rf = iota + (r * BPR + 16 * g)
      icol = ((plsc.load_gather(i_idx, [rf]) >> 8) & one) << 6
      ncol = [
          ((plsc.load_gather(n_idx, [jnp.full((16,), k, jnp.int32), rf]) >> 8)
           & one) << 6 for k in range(NUM_NEG)
      ]

      kvs = [jnp.full((16,), k, jnp.int32) for k in range(NUM_NEG)]

      def dim_body(d, carry, row=row, lrow=lrow, ucol=ucol, icol=icol,
                   ncol=ncol, kvs=kvs):
        ds = jnp.full((16,), d, jnp.int32)
        u_d = plsc.load_gather(u_loc, [lrow, ucol + ds])
        p = carry[0] + u_d * plsc.load_gather(i_rows, [row, icol + ds])
        ns = []
        for k in range(NUM_NEG):
          ns.append(carry[1 + k] + u_d *
                    plsc.load_gather(n_rows, [kvs[k], row, ncol[k] + ds]))
        return (p, *ns)

      scores = lax.fori_loop(0, DIM, dim_body, (zero,) * (1 + NUM_NEG),
                             unroll=4)
      sl = pl.ds(r * BPR + 16 * g, 16)
      pos_v[sl] = scores[0]
      nexp = jnp.exp(scores[1])
      for k in range(2, NUM_NEG + 1):
        nexp = nexp + jnp.exp(scores[k])
      nexp_v[sl] = nexp

  pltpu.sync_copy(pos_v, pos_h.at[pl.ds(base, BPW)])
  pltpu.sync_copy(nexp_v, nexp_h.at[pl.ds(base, BPW)])


_sc_scores = functools.partial(
    pl.kernel,
    mesh=plsc.VectorSubcoreMesh(core_axis_name="c", subcore_axis_name="s"),
    out_type=[
        jax.ShapeDtypeStruct((BATCH,), jnp.float32),
        jax.ShapeDtypeStruct((BATCH,), jnp.float32),
    ],
    scratch_types=[
        pltpu.VMEM((BPW,), jnp.int32),
        pltpu.VMEM((NUM_NEG, BPW), jnp.int32),
        pltpu.VMEM((BPR,), jnp.int32),
        pltpu.VMEM((NUM_NEG, BPR), jnp.int32),
        pltpu.VMEM((BPR // 2, 2 * DIM), jnp.float32),
        pltpu.VMEM((BPR, 2 * DIM), jnp.float32),
        pltpu.VMEM((NUM_NEG, BPR, 2 * DIM), jnp.float32),
        pltpu.VMEM((BPW,), jnp.float32),
        pltpu.VMEM((BPW,), jnp.float32),
        pltpu.SemaphoreType.DMA,
    ],
    compiler_params=pltpu.CompilerParams(
        needs_layout_passes=False, use_tc_tiling_on_sc=True),
)(_sc_scores_body)


def _tc_transpose_body(xT_ref, eye_ref, o_ref):
  eye = eye_ref[...]
  dn = (((1,), (1,)), ((), ()))
  for m in range(8):
    x = xT_ref[:, pl.ds(512 * m, 512)]
    xx = jnp.concatenate([x[:, 0:256], x[:, 256:512]], axis=0)
    o_ref[pl.ds(256 * m, 256), :] = lax.dot_general(
        eye, xx, dn, preferred_element_type=jnp.float32)


_N_TBLK = 245
_tc_transpose = pl.pallas_call(
    _tc_transpose_body,
    out_shape=jax.ShapeDtypeStruct((_N_TBLK * 2048, 2 * DIM), jnp.float32),
    grid=(_N_TBLK,),
    in_specs=[
        pl.BlockSpec((DIM, 4096), lambda c: (0, c)),
        pl.BlockSpec((256, 256), lambda c: (0, 0)),
    ],
    out_specs=pl.BlockSpec((2048, 2 * DIM), lambda c: (c, 0)),
)


def _tc_loss_body(pos_ref, nexp_ref, o_ref):
  pe = jnp.exp(pos_ref[...])
  ne = nexp_ref[...]
  losses = -jnp.log(pe / (pe + ne))
  o_ref[0, 0] = jnp.sum(losses) * (1.0 / BATCH)


_tc_loss = pl.pallas_call(
    _tc_loss_body,
    out_shape=jax.ShapeDtypeStruct((1, 1), jnp.float32),
    out_specs=pl.BlockSpec(memory_space=pltpu.SMEM),
)


def kernel(users, items, negatives, user_emb, item_emb):
  users = users.astype(jnp.int32)
  items = items.astype(jnp.int32)
  negatives = negatives.astype(jnp.int32)
  uembT3 = user_emb.T.reshape(DIM // 8, 8, NUM_ROWS)
  ustage = _sc_users(users, uembT3)
  iemb2 = _tc_transpose(item_emb.T, jnp.eye(256, dtype=jnp.float32))
  pos, nexp = _sc_scores(items, negatives, iemb2, ustage)
  out = _tc_loss(pos.reshape(32, 128), nexp.reshape(32, 128))
  return out[0, 0]


# speedup vs baseline: 4.4690x; 1.2020x over previous
import functools

import jax
import jax.numpy as jnp
from jax import lax
from jax.experimental import pallas as pl
from jax.experimental.pallas import tpu as pltpu
from jax.experimental.pallas import tpu_sc as plsc

DIM = 64
BATCH = 4096
NUM_ROWS = 1000000
TAIL_START = (NUM_ROWS // 128) * 128
TAIL = NUM_ROWS - TAIL_START
NUM_NEG = 8
NUM_CORES = 2
NUM_SUBCORES = 16
NUM_WORKERS = NUM_CORES * NUM_SUBCORES
BPW = BATCH // NUM_WORKERS
RPW = 2
BPR = BPW // RPW
GROUPS = BPR // 16


def _worker_id():
  return lax.axis_index("s") * NUM_CORES + lax.axis_index("c")


def _sc_users_body(users_h, uembT_h, ustage_h, idx_vm, ublk, tailbuf, u_loc,
                   sem):
  wid = _worker_id()
  base = wid * BPW

  pltpu.sync_copy(users_h.at[pl.ds(base, BPW)], idx_vm.at[pl.ds(0, BPW)])
  pltpu.sync_copy(uembT_h.at[:, :, pl.ds(TAIL_START, TAIL)], tailbuf)

  iota = lax.iota(jnp.int32, 16)
  half_a = iota >> 3
  mvec = iota & 7

  def scalar_idx(b):
    return idx_vm[pl.ds(b, 16)][0]

  def fire(b):
    j = scalar_idx(b)
    jc = jnp.minimum(j >> 7, TAIL_START // 128 - 1)
    off = pl.multiple_of(jc * 128, 128)
    p = b & 1
    for a in range(DIM // 8):
      pltpu.async_copy(uembT_h.at[a, :, pl.ds(off, 128)], ublk.at[p, a], sem)

  def drain():
    for _ in range(DIM // 8):
      pltpu.make_async_copy(uembT_h.at[0, :, pl.ds(0, 128)], ublk.at[0, 0],
                            sem).wait()

  fire(0)

  @pl.loop(0, BPW)
  def _per_index(b):
    @pl.when(b + 1 < BPW)
    def _():
      fire(b + 1)
    drain()
    j = scalar_idx(b)
    p = jnp.full((16,), b & 1, jnp.int32)
    col = jnp.full((16,), j & 127, jnp.int32)
    tcol = jnp.full((16,), jnp.maximum(j - TAIL_START, 0), jnp.int32)
    tmask = jnp.full((16,), j, jnp.int32) >= TAIL_START
    prow = jnp.full((16,), b >> 1, jnp.int32)
    pcol = iota + ((b & 1) << 6)
    for q in range(4):
      avec = half_a + 2 * q
      vn = plsc.load_gather(ublk, [p, avec, mvec, col])
      vt = plsc.load_gather(tailbuf, [avec, mvec, tcol])
      v = jnp.where(tmask, vt, vn)
      plsc.store_scatter(u_loc, [prow, pcol + 16 * q], v)

  pltpu.sync_copy(
      u_loc, ustage_h.at[pl.ds(pl.multiple_of(base // 2, 8), BPW // 2)])


_sc_users = functools.partial(
    pl.kernel,
    mesh=plsc.VectorSubcoreMesh(core_axis_name="c", subcore_axis_name="s"),
    out_type=jax.ShapeDtypeStruct((BATCH // 2, 2 * DIM), jnp.float32),
    scratch_types=[
        pltpu.VMEM((BPW + 16,), jnp.int32),
        pltpu.VMEM((2, DIM // 8, 8, 128), jnp.float32),
        pltpu.VMEM((DIM // 8, 8, TAIL), jnp.float32),
        pltpu.VMEM((BPW // 2, 2 * DIM), jnp.float32),
        pltpu.SemaphoreType.DMA,
    ],
    compiler_params=pltpu.CompilerParams(
        needs_layout_passes=False, use_tc_tiling_on_sc=True),
)(_sc_users_body)


def _sc_scores_body(items_h, negs_h, iemb2_h, ustage_h, pos_h, nexp_h,
                    i_idx, n_idx, i2, n2, u_loc, i_rows, n_rows, pos_v,
                    nexp_v, sem):
  wid = _worker_id()
  base = wid * BPW

  pltpu.sync_copy(items_h.at[pl.ds(base, BPW)], i_idx)
  for k in range(NUM_NEG):
    pltpu.sync_copy(negs_h.at[pl.ds(k * BATCH + base, BPW)], n_idx.at[k])

  iota = lax.iota(jnp.int32, 16)
  zero = jnp.zeros((16,), jnp.float32)
  one = jnp.full((16,), 1, jnp.int32)

  for r in range(RPW):
    for c in range(GROUPS):
      v = i_idx[pl.ds(r * BPR + 16 * c, 16)]
      i2[pl.ds(16 * c, 16)] = ((v >> 9) << 8) | (v & 255)
      for k in range(NUM_NEG):
        v = n_idx[k, pl.ds(r * BPR + 16 * c, 16)]
        n2[k, pl.ds(16 * c, 16)] = ((v >> 9) << 8) | (v & 255)

    pltpu.sync_copy(
        ustage_h.at[pl.ds(
            pl.multiple_of(base // 2 + r * (BPR // 2), 8), BPR // 2)], u_loc)
    cps = [pltpu.async_copy(iemb2_h.at[i2], i_rows, sem)]
    for k in range(NUM_NEG):
      cps.append(pltpu.async_copy(iemb2_h.at[n2.at[k]], n_rows.at[k], sem))
    for cp in cps:
      cp.wait()

    for g in range(GROUPS):
      row = iota + 16 * g
      lrow = row >> 1
      ucol = (row & one) << 6
      rf = iota + (r * BPR + 16 * g)
      icol = ((plsc.load_gather(i_idx, [rf]) >> 8) & one) << 6
      ncol = [
          ((plsc.load_gather(n_idx, [jnp.full((16,), k, jnp.int32), rf]) >> 8)
           & one) << 6 for k in range(NUM_NEG)
      ]

      kvs = [jnp.full((16,), k, jnp.int32) for k in range(NUM_NEG)]

      def dim_body(d, carry, row=row, lrow=lrow, ucol=ucol, icol=icol,
                   ncol=ncol, kvs=kvs):
        ds = jnp.full((16,), d, jnp.int32)
        u_d = plsc.load_gather(u_loc, [lrow, ucol + ds])
        p = carry[0] + u_d * plsc.load_gather(i_rows, [row, icol + ds])
        ns = []
        for k in range(NUM_NEG):
          ns.append(carry[1 + k] + u_d *
                    plsc.load_gather(n_rows, [kvs[k], row, ncol[k] + ds]))
        return (p, *ns)

      scores = lax.fori_loop(0, DIM, dim_body, (zero,) * (1 + NUM_NEG),
                             unroll=4)
      sl = pl.ds(r * BPR + 16 * g, 16)
      pos_v[sl] = scores[0]
      nexp = jnp.exp(scores[1])
      for k in range(2, NUM_NEG + 1):
        nexp = nexp + jnp.exp(scores[k])
      nexp_v[sl] = nexp

  pltpu.sync_copy(pos_v, pos_h.at[pl.ds(base, BPW)])
  pltpu.sync_copy(nexp_v, nexp_h.at[pl.ds(base, BPW)])


_sc_scores = functools.partial(
    pl.kernel,
    mesh=plsc.VectorSubcoreMesh(core_axis_name="c", subcore_axis_name="s"),
    out_type=[
        jax.ShapeDtypeStruct((BATCH,), jnp.float32),
        jax.ShapeDtypeStruct((BATCH,), jnp.float32),
    ],
    scratch_types=[
        pltpu.VMEM((BPW,), jnp.int32),
        pltpu.VMEM((NUM_NEG, BPW), jnp.int32),
        pltpu.VMEM((BPR,), jnp.int32),
        pltpu.VMEM((NUM_NEG, BPR), jnp.int32),
        pltpu.VMEM((BPR // 2, 2 * DIM), jnp.float32),
        pltpu.VMEM((BPR, 2 * DIM), jnp.float32),
        pltpu.VMEM((NUM_NEG, BPR, 2 * DIM), jnp.float32),
        pltpu.VMEM((BPW,), jnp.float32),
        pltpu.VMEM((BPW,), jnp.float32),
        pltpu.SemaphoreType.DMA,
    ],
    compiler_params=pltpu.CompilerParams(
        needs_layout_passes=False, use_tc_tiling_on_sc=True),
)(_sc_scores_body)


def _tc_transpose_body(xT_ref, eye_ref, o_ref):
  eye = eye_ref[...]
  dn = (((1,), (1,)), ((), ()))
  for m in range(16):
    x = xT_ref[:, pl.ds(512 * m, 512)]
    xx = jnp.concatenate([x[:, 0:256], x[:, 256:512]], axis=0)
    o_ref[pl.ds(256 * m, 256), :] = lax.dot_general(
        eye, xx, dn, preferred_element_type=jnp.float32)


_N_TBLK = 123
_tc_transpose = pl.pallas_call(
    _tc_transpose_body,
    out_shape=jax.ShapeDtypeStruct((_N_TBLK * 4096, 2 * DIM), jnp.float32),
    grid=(_N_TBLK,),
    in_specs=[
        pl.BlockSpec((DIM, 8192), lambda c: (0, c)),
        pl.BlockSpec((256, 256), lambda c: (0, 0)),
    ],
    out_specs=pl.BlockSpec((4096, 2 * DIM), lambda c: (c, 0)),
)


def _tc_loss_body(pos_ref, nexp_ref, o_ref):
  pe = jnp.exp(pos_ref[...])
  ne = nexp_ref[...]
  losses = -jnp.log(pe / (pe + ne))
  o_ref[0, 0] = jnp.sum(losses) * (1.0 / BATCH)


_tc_loss = pl.pallas_call(
    _tc_loss_body,
    out_shape=jax.ShapeDtypeStruct((1, 1), jnp.float32),
    out_specs=pl.BlockSpec(memory_space=pltpu.SMEM),
)


def kernel(users, items, negatives, user_emb, item_emb):
  users = users.astype(jnp.int32)
  items = items.astype(jnp.int32)
  negatives = negatives.astype(jnp.int32)
  uembT3 = user_emb.T.reshape(DIM // 8, 8, NUM_ROWS)
  ustage = _sc_users(users, uembT3)
  iemb2 = _tc_transpose(item_emb.T, jnp.eye(256, dtype=jnp.float32))
  pos, nexp = _sc_scores(items, negatives, iemb2, ustage)
  out = _tc_loss(pos.reshape(32, 128), nexp.reshape(32, 128))
  return out[0, 0]


# speedup vs baseline: 4.7159x; 1.0553x over previous
import functools

import jax
import jax.numpy as jnp
from jax import lax
from jax.experimental import pallas as pl
from jax.experimental.pallas import tpu as pltpu
from jax.experimental.pallas import tpu_sc as plsc

DIM = 64
BATCH = 4096
NUM_ROWS = 1000000
TAIL_START = (NUM_ROWS // 128) * 128
TAIL = NUM_ROWS - TAIL_START
NUM_NEG = 8
NUM_CORES = 2
NUM_SUBCORES = 16
NUM_WORKERS = NUM_CORES * NUM_SUBCORES
BPW = BATCH // NUM_WORKERS
RPW = 2
BPR = BPW // RPW
GROUPS = BPR // 16


def _worker_id():
  return lax.axis_index("s") * NUM_CORES + lax.axis_index("c")


def _sc_users_body(users_h, uembT_h, ustage_h, idx_vm, ublk, tailbuf, u_loc,
                   sem):
  wid = _worker_id()
  base = wid * BPW

  pltpu.sync_copy(users_h.at[pl.ds(base, BPW)], idx_vm.at[pl.ds(0, BPW)])
  pltpu.sync_copy(uembT_h.at[:, :, pl.ds(TAIL_START, TAIL)], tailbuf)

  iota = lax.iota(jnp.int32, 16)
  half_a = iota >> 3
  mvec = iota & 7

  def scalar_idx(b):
    return idx_vm[pl.ds(b, 16)][0]

  def fire(b):
    j = scalar_idx(b)
    jc = jnp.minimum(j >> 7, TAIL_START // 128 - 1)
    off = pl.multiple_of(jc * 128, 128)
    p = b & 1
    for a in range(DIM // 8):
      pltpu.async_copy(uembT_h.at[a, :, pl.ds(off, 128)], ublk.at[p, a], sem)

  def drain():
    for _ in range(DIM // 8):
      pltpu.make_async_copy(uembT_h.at[0, :, pl.ds(0, 128)], ublk.at[0, 0],
                            sem).wait()

  fire(0)

  @pl.loop(0, BPW)
  def _per_index(b):
    @pl.when(b + 1 < BPW)
    def _():
      fire(b + 1)
    drain()
    j = scalar_idx(b)
    p = jnp.full((16,), b & 1, jnp.int32)
    col = jnp.full((16,), j & 127, jnp.int32)
    tcol = jnp.full((16,), jnp.maximum(j - TAIL_START, 0), jnp.int32)
    tmask = jnp.full((16,), j, jnp.int32) >= TAIL_START
    prow = jnp.full((16,), b >> 1, jnp.int32)
    pcol = iota + ((b & 1) << 6)
    for q in range(4):
      avec = half_a + 2 * q
      vn = plsc.load_gather(ublk, [p, avec, mvec, col])
      vt = plsc.load_gather(tailbuf, [avec, mvec, tcol])
      v = jnp.where(tmask, vt, vn)
      plsc.store_scatter(u_loc, [prow, pcol + 16 * q], v)

  pltpu.sync_copy(
      u_loc, ustage_h.at[pl.ds(pl.multiple_of(base // 2, 8), BPW // 2)])


_sc_users = functools.partial(
    pl.kernel,
    mesh=plsc.VectorSubcoreMesh(core_axis_name="c", subcore_axis_name="s"),
    out_type=jax.ShapeDtypeStruct((BATCH // 2, 2 * DIM), jnp.float32),
    scratch_types=[
        pltpu.VMEM((BPW + 16,), jnp.int32),
        pltpu.VMEM((2, DIM // 8, 8, 128), jnp.float32),
        pltpu.VMEM((DIM // 8, 8, TAIL), jnp.float32),
        pltpu.VMEM((BPW // 2, 2 * DIM), jnp.float32),
        pltpu.SemaphoreType.DMA,
    ],
    compiler_params=pltpu.CompilerParams(
        needs_layout_passes=False, use_tc_tiling_on_sc=True),
)(_sc_users_body)


def _sc_scores_body(items_h, negs_h, iemb2_h, ustage_h, pos_h, nexp_h,
                    i_idx, n_idx, i2, n2, u_loc, i_rows, n_rows, pos_v,
                    nexp_v, sem):
  wid = _worker_id()
  base = wid * BPW

  pltpu.sync_copy(items_h.at[pl.ds(base, BPW)], i_idx)
  for k in range(NUM_NEG):
    pltpu.sync_copy(negs_h.at[pl.ds(k * BATCH + base, BPW)], n_idx.at[k])

  iota = lax.iota(jnp.int32, 16)
  zero = jnp.zeros((16,), jnp.float32)
  one = jnp.full((16,), 1, jnp.int32)

  for r in range(RPW):
    for c in range(GROUPS):
      v = i_idx[pl.ds(r * BPR + 16 * c, 16)]
      i2[pl.ds(16 * c, 16)] = ((v >> 9) << 8) | (v & 255)
      for k in range(NUM_NEG):
        v = n_idx[k, pl.ds(r * BPR + 16 * c, 16)]
        n2[k, pl.ds(16 * c, 16)] = ((v >> 9) << 8) | (v & 255)

    pltpu.sync_copy(
        ustage_h.at[pl.ds(
            pl.multiple_of(base // 2 + r * (BPR // 2), 8), BPR // 2)], u_loc)
    cps = [pltpu.async_copy(iemb2_h.at[i2], i_rows, sem)]
    for k in range(NUM_NEG):
      cps.append(pltpu.async_copy(iemb2_h.at[n2.at[k]], n_rows.at[k], sem))
    for cp in cps:
      cp.wait()

    for g in range(GROUPS):
      row = iota + 16 * g
      lrow = row >> 1
      ucol = (row & one) << 6
      rf = iota + (r * BPR + 16 * g)
      icol = ((plsc.load_gather(i_idx, [rf]) >> 8) & one) << 6
      ncol = [
          ((plsc.load_gather(n_idx, [jnp.full((16,), k, jnp.int32), rf]) >> 8)
           & one) << 6 for k in range(NUM_NEG)
      ]

      kvs = [jnp.full((16,), k, jnp.int32) for k in range(NUM_NEG)]

      def dim_body(d, carry, row=row, lrow=lrow, ucol=ucol, icol=icol,
                   ncol=ncol, kvs=kvs):
        ds = jnp.full((16,), d, jnp.int32)
        u_d = plsc.load_gather(u_loc, [lrow, ucol + ds])
        p = carry[0] + u_d * plsc.load_gather(i_rows, [row, icol + ds])
        ns = []
        for k in range(NUM_NEG):
          ns.append(carry[1 + k] + u_d *
                    plsc.load_gather(n_rows, [kvs[k], row, ncol[k] + ds]))
        return (p, *ns)

      scores = lax.fori_loop(0, DIM, dim_body, (zero,) * (1 + NUM_NEG),
                             unroll=4)
      sl = pl.ds(r * BPR + 16 * g, 16)
      pos_v[sl] = scores[0]
      nexp = jnp.exp(scores[1])
      for k in range(2, NUM_NEG + 1):
        nexp = nexp + jnp.exp(scores[k])
      nexp_v[sl] = nexp

  pltpu.sync_copy(pos_v, pos_h.at[pl.ds(base, BPW)])
  pltpu.sync_copy(nexp_v, nexp_h.at[pl.ds(base, BPW)])


_sc_scores = functools.partial(
    pl.kernel,
    mesh=plsc.VectorSubcoreMesh(core_axis_name="c", subcore_axis_name="s"),
    out_type=[
        jax.ShapeDtypeStruct((BATCH,), jnp.float32),
        jax.ShapeDtypeStruct((BATCH,), jnp.float32),
    ],
    scratch_types=[
        pltpu.VMEM((BPW,), jnp.int32),
        pltpu.VMEM((NUM_NEG, BPW), jnp.int32),
        pltpu.VMEM((BPR,), jnp.int32),
        pltpu.VMEM((NUM_NEG, BPR), jnp.int32),
        pltpu.VMEM((BPR // 2, 2 * DIM), jnp.float32),
        pltpu.VMEM((BPR, 2 * DIM), jnp.float32),
        pltpu.VMEM((NUM_NEG, BPR, 2 * DIM), jnp.float32),
        pltpu.VMEM((BPW,), jnp.float32),
        pltpu.VMEM((BPW,), jnp.float32),
        pltpu.SemaphoreType.DMA,
    ],
    compiler_params=pltpu.CompilerParams(
        needs_layout_passes=False, use_tc_tiling_on_sc=True),
)(_sc_scores_body)


def _tc_transpose_body(xT_ref, eye_ref, o_ref):
  eye = eye_ref[...]
  dn = (((1,), (1,)), ((), ()))
  for m in range(32):
    x = xT_ref[:, pl.ds(512 * m, 512)]
    xx = jnp.concatenate([x[:, 0:256], x[:, 256:512]], axis=0)
    o_ref[pl.ds(256 * m, 256), :] = lax.dot_general(
        eye, xx, dn, preferred_element_type=jnp.float32)


_N_TBLK = 62
_tc_transpose = pl.pallas_call(
    _tc_transpose_body,
    out_shape=jax.ShapeDtypeStruct((_N_TBLK * 8192, 2 * DIM), jnp.float32),
    grid=(_N_TBLK,),
    in_specs=[
        pl.BlockSpec((DIM, 16384), lambda c: (0, c)),
        pl.BlockSpec((256, 256), lambda c: (0, 0)),
    ],
    out_specs=pl.BlockSpec((8192, 2 * DIM), lambda c: (c, 0)),
)


def _tc_loss_body(pos_ref, nexp_ref, o_ref):
  pe = jnp.exp(pos_ref[...])
  ne = nexp_ref[...]
  losses = -jnp.log(pe / (pe + ne))
  o_ref[0, 0] = jnp.sum(losses) * (1.0 / BATCH)


_tc_loss = pl.pallas_call(
    _tc_loss_body,
    out_shape=jax.ShapeDtypeStruct((1, 1), jnp.float32),
    out_specs=pl.BlockSpec(memory_space=pltpu.SMEM),
)


def kernel(users, items, negatives, user_emb, item_emb):
  users = users.astype(jnp.int32)
  items = items.astype(jnp.int32)
  negatives = negatives.astype(jnp.int32)
  uembT3 = user_emb.T.reshape(DIM // 8, 8, NUM_ROWS)
  ustage = _sc_users(users, uembT3)
  iemb2 = _tc_transpose(item_emb.T, jnp.eye(256, dtype=jnp.float32))
  pos, nexp = _sc_scores(items, negatives, iemb2, ustage)
  out = _tc_loss(pos.reshape(32, 128), nexp.reshape(32, 128))
  return out[0, 0]


# speedup vs baseline: 4.8063x; 1.0192x over previous
import functools

import jax
import jax.numpy as jnp
from jax import lax
from jax.experimental import pallas as pl
from jax.experimental.pallas import tpu as pltpu
from jax.experimental.pallas import tpu_sc as plsc

DIM = 64
BATCH = 4096
NUM_ROWS = 1000000
TAIL_START = (NUM_ROWS // 128) * 128
TAIL = NUM_ROWS - TAIL_START
NUM_NEG = 8
NUM_CORES = 2
NUM_SUBCORES = 16
NUM_WORKERS = NUM_CORES * NUM_SUBCORES
BPW = BATCH // NUM_WORKERS
RPW = 2
BPR = BPW // RPW
GROUPS = BPR // 16


def _worker_id():
  return lax.axis_index("s") * NUM_CORES + lax.axis_index("c")


def _sc_users_body(users_h, uembT_h, ustage_h, idx_vm, ublk, tailbuf, u_loc,
                   sem):
  wid = _worker_id()
  base = wid * BPW

  pltpu.sync_copy(users_h.at[pl.ds(base, BPW)], idx_vm.at[pl.ds(0, BPW)])
  pltpu.sync_copy(uembT_h.at[:, :, pl.ds(TAIL_START, TAIL)], tailbuf)

  iota = lax.iota(jnp.int32, 16)
  half_a = iota >> 3
  mvec = iota & 7

  def scalar_idx(b):
    return idx_vm[pl.ds(b, 16)][0]

  def fire(b):
    j = scalar_idx(b)
    jc = jnp.minimum(j >> 7, TAIL_START // 128 - 1)
    off = pl.multiple_of(jc * 128, 128)
    p = b & 1
    for a in range(DIM // 8):
      pltpu.async_copy(uembT_h.at[a, :, pl.ds(off, 128)], ublk.at[p, a], sem)

  def drain():
    for _ in range(DIM // 8):
      pltpu.make_async_copy(uembT_h.at[0, :, pl.ds(0, 128)], ublk.at[0, 0],
                            sem).wait()

  fire(0)

  @pl.loop(0, BPW)
  def _per_index(b):
    @pl.when(b + 1 < BPW)
    def _():
      fire(b + 1)
    drain()
    j = scalar_idx(b)
    p = jnp.full((16,), b & 1, jnp.int32)
    col = jnp.full((16,), j & 127, jnp.int32)
    tcol = jnp.full((16,), jnp.maximum(j - TAIL_START, 0), jnp.int32)
    tmask = jnp.full((16,), j, jnp.int32) >= TAIL_START
    prow = jnp.full((16,), b >> 1, jnp.int32)
    pcol = iota + ((b & 1) << 6)
    for q in range(4):
      avec = half_a + 2 * q
      vn = plsc.load_gather(ublk, [p, avec, mvec, col])
      vt = plsc.load_gather(tailbuf, [avec, mvec, tcol])
      v = jnp.where(tmask, vt, vn)
      plsc.store_scatter(u_loc, [prow, pcol + 16 * q], v)

  pltpu.sync_copy(
      u_loc, ustage_h.at[pl.ds(pl.multiple_of(base // 2, 8), BPW // 2)])


_sc_users = functools.partial(
    pl.kernel,
    mesh=plsc.VectorSubcoreMesh(core_axis_name="c", subcore_axis_name="s"),
    out_type=jax.ShapeDtypeStruct((BATCH // 2, 2 * DIM), jnp.float32),
    scratch_types=[
        pltpu.VMEM((BPW + 16,), jnp.int32),
        pltpu.VMEM((2, DIM // 8, 8, 128), jnp.float32),
        pltpu.VMEM((DIM // 8, 8, TAIL), jnp.float32),
        pltpu.VMEM((BPW // 2, 2 * DIM), jnp.float32),
        pltpu.SemaphoreType.DMA,
    ],
    compiler_params=pltpu.CompilerParams(
        needs_layout_passes=False, use_tc_tiling_on_sc=True),
)(_sc_users_body)


def _sc_scores_body(items_h, negs_h, iemb2_h, ustage_h, pos_h, nexp_h,
                    i_idx, n_idx, i2, n2, u_loc, i_rows, n_rows, pos_v,
                    nexp_v, sem):
  wid = _worker_id()
  base = wid * BPW

  pltpu.sync_copy(items_h.at[pl.ds(base, BPW)], i_idx)
  for k in range(NUM_NEG):
    pltpu.sync_copy(negs_h.at[pl.ds(k * BATCH + base, BPW)], n_idx.at[k])

  iota = lax.iota(jnp.int32, 16)
  zero = jnp.zeros((16,), jnp.float32)
  one = jnp.full((16,), 1, jnp.int32)

  for r in range(RPW):
    for c in range(GROUPS):
      v = i_idx[pl.ds(r * BPR + 16 * c, 16)]
      i2[pl.ds(16 * c, 16)] = ((v >> 9) << 8) | (v & 255)
      for k in range(NUM_NEG):
        v = n_idx[k, pl.ds(r * BPR + 16 * c, 16)]
        n2[k, pl.ds(16 * c, 16)] = ((v >> 9) << 8) | (v & 255)

    pltpu.sync_copy(
        ustage_h.at[pl.ds(
            pl.multiple_of(base // 2 + r * (BPR // 2), 8), BPR // 2)], u_loc)
    cps = [pltpu.async_copy(iemb2_h.at[i2], i_rows, sem)]
    for k in range(NUM_NEG):
      cps.append(pltpu.async_copy(iemb2_h.at[n2.at[k]], n_rows.at[k], sem))
    for cp in cps:
      cp.wait()

    for g in range(GROUPS):
      row = iota + 16 * g
      lrow = row >> 1
      ucol = (row & one) << 6
      rf = iota + (r * BPR + 16 * g)
      icol = ((plsc.load_gather(i_idx, [rf]) >> 8) & one) << 6
      ncol = [
          ((plsc.load_gather(n_idx, [jnp.full((16,), k, jnp.int32), rf]) >> 8)
           & one) << 6 for k in range(NUM_NEG)
      ]

      kvs = [jnp.full((16,), k, jnp.int32) for k in range(NUM_NEG)]

      def dim_body(d, carry, row=row, lrow=lrow, ucol=ucol, icol=icol,
                   ncol=ncol, kvs=kvs):
        ds = jnp.full((16,), d, jnp.int32)
        u_d = plsc.load_gather(u_loc, [lrow, ucol + ds])
        p = carry[0] + u_d * plsc.load_gather(i_rows, [row, icol + ds])
        ns = []
        for k in range(NUM_NEG):
          ns.append(carry[1 + k] + u_d *
                    plsc.load_gather(n_rows, [kvs[k], row, ncol[k] + ds]))
        return (p, *ns)

      scores = lax.fori_loop(0, DIM, dim_body, (zero,) * (1 + NUM_NEG),
                             unroll=4)
      sl = pl.ds(r * BPR + 16 * g, 16)
      pos_v[sl] = scores[0]
      nexp = jnp.exp(scores[1])
      for k in range(2, NUM_NEG + 1):
        nexp = nexp + jnp.exp(scores[k])
      nexp_v[sl] = nexp

  pltpu.sync_copy(pos_v, pos_h.at[pl.ds(base, BPW)])
  pltpu.sync_copy(nexp_v, nexp_h.at[pl.ds(base, BPW)])


_sc_scores = functools.partial(
    pl.kernel,
    mesh=plsc.VectorSubcoreMesh(core_axis_name="c", subcore_axis_name="s"),
    out_type=[
        jax.ShapeDtypeStruct((BATCH,), jnp.float32),
        jax.ShapeDtypeStruct((BATCH,), jnp.float32),
    ],
    scratch_types=[
        pltpu.VMEM((BPW,), jnp.int32),
        pltpu.VMEM((NUM_NEG, BPW), jnp.int32),
        pltpu.VMEM((BPR,), jnp.int32),
        pltpu.VMEM((NUM_NEG, BPR), jnp.int32),
        pltpu.VMEM((BPR // 2, 2 * DIM), jnp.float32),
        pltpu.VMEM((BPR, 2 * DIM), jnp.float32),
        pltpu.VMEM((NUM_NEG, BPR, 2 * DIM), jnp.float32),
        pltpu.VMEM((BPW,), jnp.float32),
        pltpu.VMEM((BPW,), jnp.float32),
        pltpu.SemaphoreType.DMA,
    ],
    compiler_params=pltpu.CompilerParams(
        needs_layout_passes=False, use_tc_tiling_on_sc=True),
)(_sc_scores_body)


def _tc_transpose_body(xT_ref, eye_ref, o_ref):
  eye = eye_ref[...]
  dn = (((1,), (1,)), ((), ()))
  for m in range(64):
    x = xT_ref[:, pl.ds(512 * m, 512)]
    xx = jnp.concatenate([x[:, 0:256], x[:, 256:512]], axis=0)
    o_ref[pl.ds(256 * m, 256), :] = lax.dot_general(
        eye, xx, dn, preferred_element_type=jnp.float32)


_N_TBLK = 31
_tc_transpose = pl.pallas_call(
    _tc_transpose_body,
    out_shape=jax.ShapeDtypeStruct((_N_TBLK * 16384, 2 * DIM), jnp.float32),
    grid=(_N_TBLK,),
    in_specs=[
        pl.BlockSpec((DIM, 32768), lambda c: (0, c)),
        pl.BlockSpec((256, 256), lambda c: (0, 0)),
    ],
    out_specs=pl.BlockSpec((16384, 2 * DIM), lambda c: (c, 0)),
)


def _tc_loss_body(pos_ref, nexp_ref, o_ref):
  pe = jnp.exp(pos_ref[...])
  ne = nexp_ref[...]
  losses = -jnp.log(pe / (pe + ne))
  o_ref[0, 0] = jnp.sum(losses) * (1.0 / BATCH)


_tc_loss = pl.pallas_call(
    _tc_loss_body,
    out_shape=jax.ShapeDtypeStruct((1, 1), jnp.float32),
    out_specs=pl.BlockSpec(memory_space=pltpu.SMEM),
)


def kernel(users, items, negatives, user_emb, item_emb):
  users = users.astype(jnp.int32)
  items = items.astype(jnp.int32)
  negatives = negatives.astype(jnp.int32)
  uembT3 = user_emb.T.reshape(DIM // 8, 8, NUM_ROWS)
  ustage = _sc_users(users, uembT3)
  iemb2 = _tc_transpose(item_emb.T, jnp.eye(256, dtype=jnp.float32))
  pos, nexp = _sc_scores(items, negatives, iemb2, ustage)
  out = _tc_loss(pos.reshape(32, 128), nexp.reshape(32, 128))
  return out[0, 0]
